# baseline (speedup 1.0000x reference)
import functools

import jax
import jax.numpy as jnp
from jax.experimental import pallas as pl
from jax.experimental.pallas import tpu as pltpu

_EPS = 1e-5
_F32 = jnp.float32


def _gemm_body(*refs, in_bn, bias, stats):
    it = iter(refs)
    a_ref = next(it)
    w_ref = next(it)
    if in_bn:
        nsc_ref = next(it)
        nsh_ref = next(it)
    if bias:
        b_ref = next(it)
    o_ref = next(it)
    if stats:
        st_ref = next(it)

    a = a_ref[...]
    if in_bn:
        a = jnp.maximum(a * nsc_ref[0:1, :] + nsh_ref[0:1, :], 0.0)
    o = jnp.dot(a.astype(jnp.bfloat16), w_ref[...].astype(jnp.bfloat16),
                preferred_element_type=_F32)
    if bias:
        o = o + b_ref[0:1, :]
    o_ref[...] = o
    if stats:
        n = o.shape[1]

        @pl.when(pl.program_id(0) == 0)
        def _init():
            st_ref[...] = jnp.zeros_like(st_ref)

        upd = jnp.concatenate(
            [
                jnp.sum(o, axis=0)[None, :],
                jnp.sum(o * o, axis=0)[None, :],
                jnp.zeros((6, n), _F32),
            ],
            axis=0,
        )
        st_ref[...] += upd


def _gemm(a, w, nsc=None, nsh=None, b=None, stats=False, tm=512):
    m, k = a.shape
    n = w.shape[1]
    tm = min(tm, m)
    in_bn = nsc is not None
    bias = b is not None

    in_specs = [
        pl.BlockSpec((tm, k), lambda i: (i, 0)),
        pl.BlockSpec((k, n), lambda i: (0, 0)),
    ]
    operands = [a, w]
    if in_bn:
        in_specs += [
            pl.BlockSpec((1, k), lambda i: (0, 0)),
            pl.BlockSpec((1, k), lambda i: (0, 0)),
        ]
        operands += [nsc.reshape(1, k), nsh.reshape(1, k)]
    if bias:
        in_specs.append(pl.BlockSpec((1, n), lambda i: (0, 0)))
        operands.append(b.reshape(1, n))

    out_shape = [jax.ShapeDtypeStruct((m, n), _F32)]
    out_specs = [pl.BlockSpec((tm, n), lambda i: (i, 0))]
    if stats:
        out_shape.append(jax.ShapeDtypeStruct((8, n), _F32))
        out_specs.append(pl.BlockSpec((8, n), lambda i: (0, 0)))

    res = pl.pallas_call(
        functools.partial(_gemm_body, in_bn=in_bn, bias=bias, stats=stats),
        grid=(m // tm,),
        in_specs=in_specs,
        out_specs=out_specs,
        out_shape=out_shape,
        compiler_params=pltpu.CompilerParams(
            dimension_semantics=("arbitrary",),
        ),
    )(*operands)
    return res if stats else res[0]


def _vq_body(d_ref, idx_ref):
    scores = d_ref[...]
    k = scores.shape[1]
    m = jnp.min(scores, axis=1, keepdims=True)
    ki = jax.lax.broadcasted_iota(jnp.int32, scores.shape, 1)
    idx = jnp.min(jnp.where(scores == m, ki, k), axis=1, keepdims=True)
    idx_ref[...] = jnp.broadcast_to(idx, idx_ref.shape)


def _vq(d):
    m, k = d.shape
    idx8 = pl.pallas_call(
        _vq_body,
        in_specs=[pl.BlockSpec((m, k), lambda: (0, 0))],
        out_specs=pl.BlockSpec((m, 128), lambda: (0, 0)),
        out_shape=jax.ShapeDtypeStruct((m, 128), jnp.int32),
    )(d)
    return idx8[:, 0]


def _bn_fin(st, g, be, n, fold=1):
    h = g.shape[0]
    s, sq = st[0], st[1]
    if fold > 1:
        s = s.reshape(fold, h).sum(axis=0)
        sq = sq.reshape(fold, h).sum(axis=0)
    mean = s / n
    var = sq / n - mean * mean
    sc = g * jax.lax.rsqrt(var + _EPS)
    sh = be - mean * sc
    return sc, sh


def _tconv(x, w, s):
    return jax.lax.conv_transpose(x, w, (s,), 'VALID',
                                  dimension_numbers=('NCH', 'IOH', 'NCH'))


def _bnorm(x, g, b):
    m = jnp.mean(x, axis=(0, 2), keepdims=True)
    v = jnp.var(x, axis=(0, 2), keepdims=True)
    return g[None, :, None] * (x - m) / jnp.sqrt(v + _EPS) + b[None, :, None]


def kernel(x, W_in, b_in, W1, g1, be1, W2, g2, be2, W3, g3, be3, codebook,
           T1, g4, be4, T2, g5, be5, T3, g6, be6, W_out, b_out):
    B, T, D = x.shape
    H = W_in.shape[1]

    W1r = jnp.transpose(W1, (2, 1, 0)).reshape(5 * H, H)
    W2r = jnp.transpose(W2, (2, 1, 0)).reshape(3 * H, H)
    W3r = jnp.transpose(W3, (2, 1, 0)).reshape(2 * H, H)
    T3r = jnp.transpose(T3, (0, 2, 1))[:, ::-1, :].reshape(H, 5 * H)

    nch = lambda m, L: jnp.transpose(m.reshape(B, L, H), (0, 2, 1))
    rows = lambda a: jnp.transpose(a, (0, 2, 1)).reshape(-1, H)

    h = _gemm(x.reshape(B * T, D), W_in, b=b_in)

    o1 = _gemm(h.reshape(B * 24, 5 * H), W1r)
    y1e = jax.nn.relu(_bnorm(nch(o1, 24), g1, be1))

    o2 = _gemm(rows(y1e).reshape(B * 8, 3 * H), W2r)
    y2e = jax.nn.relu(_bnorm(nch(o2, 8), g2, be2))

    o3 = _gemm(rows(y2e).reshape(B * 4, 2 * H), W3r)
    z_e = jax.nn.relu(_bnorm(nch(o3, 4), g3, be3))

    flat = rows(z_e)
    d = (jnp.sum(flat ** 2, axis=1, keepdims=True)
         - 2.0 * flat @ codebook.T
         + jnp.sum(codebook ** 2, axis=1)[None, :])
    idx = _vq(d)
    q = jnp.take(codebook, idx, axis=0)
    q = jnp.transpose(q.reshape(B, 4, H), (0, 2, 1))
    z_q = z_e + jax.lax.stop_gradient(q - z_e)

    y1 = jax.nn.relu(_bnorm(_tconv(z_q, T1, 2), g4, be4))
    y2 = jax.nn.relu(_bnorm(_tconv(y1, T2, 3), g5, be5))

    o6, st6 = _gemm(rows(y2), T3r, stats=True)
    sc6, sh6 = _bn_fin(st6, g6, be6, B * T, fold=5)
    y = _gemm(o6.reshape(B * T, H), W_out,
              nsc=sc6, nsh=sh6, b=b_out)
    return y.reshape(B, T, D)
